# baseline (speedup 1.0000x reference)
import functools

import jax
import jax.numpy as jnp
from jax import lax
from jax.experimental import pallas as pl
from jax.experimental.pallas import tpu as pltpu
from jax.experimental.pallas import tpu_sc as plsc

VOCAB = 1_000_000
EMBED = 64
BATCH = 4096
SEQ = 200

NC, NS = 2, 16
NW = NC * NS
ROWS_PER_W = BATCH // NW

MV_ROWS = VOCAB // 2
MV_BLK = 4000


def _matvec_body(t_ref, w_ref, b_ref, o_ref):
    o_ref[...] = (
        jnp.dot(t_ref[...], w_ref[...], preferred_element_type=jnp.float32)
        + b_ref[0, 0]
    )


def _tw_matvec(table2, wtile, b2):
    return pl.pallas_call(
        _matvec_body,
        grid=(MV_ROWS // MV_BLK,),
        in_specs=[
            pl.BlockSpec((MV_BLK, 128), lambda i: (i, 0)),
            pl.BlockSpec((128, 2), lambda i: (0, 0)),
            pl.BlockSpec((1, 1), lambda i: (0, 0)),
        ],
        out_specs=pl.BlockSpec((MV_BLK, 2), lambda i: (i, 0)),
        out_shape=jax.ShapeDtypeStruct((MV_ROWS, 2), jnp.float32),
    )(table2, wtile, b2)


def _transpose_body(x_ref, o_ref):
    o_ref[0] = jnp.swapaxes(x_ref[0], 0, 1)


def _idx_transpose(x3):
    return pl.pallas_call(
        _transpose_body,
        grid=(NW,),
        in_specs=[pl.BlockSpec((1, ROWS_PER_W, SEQ), lambda i: (i, 0, 0))],
        out_specs=pl.BlockSpec((1, SEQ, ROWS_PER_W), lambda i: (i, 0, 0)),
        out_shape=jax.ShapeDtypeStruct((NW, SEQ, ROWS_PER_W), jnp.int32),
    )(x3)


_SC_MESH = plsc.VectorSubcoreMesh(core_axis_name="c", subcore_axis_name="s")


@functools.partial(
    pl.kernel,
    out_type=jax.ShapeDtypeStruct((BATCH,), jnp.float32),
    mesh=_SC_MESH,
    scratch_types=[
        pltpu.VMEM((SEQ * ROWS_PER_W,), jnp.int32),
        pltpu.VMEM((SEQ * ROWS_PER_W,), jnp.float32),
        pltpu.VMEM((ROWS_PER_W,), jnp.float32),
        pltpu.SemaphoreType.DMA,
    ],
)
def _sc_pool(xr_hbm, tw_hbm, out_hbm, idx_v, vals_v, res_v, sem):
    wid = lax.axis_index("s") * NC + lax.axis_index("c")
    pltpu.sync_copy(xr_hbm.at[wid], idx_v)
    pltpu.async_copy(tw_hbm.at[idx_v], vals_v, sem).wait()

    ngrp = ROWS_PER_W // 16

    def seq_body(j, accs):
        base = j * ROWS_PER_W
        return tuple(
            accs[g] + vals_v[pl.ds(base + g * 16, 16)] for g in range(ngrp)
        )

    accs = lax.fori_loop(
        0, SEQ, seq_body, tuple(jnp.zeros((16,), jnp.float32) for _ in range(ngrp))
    )
    inv = jnp.float32(1.0 / SEQ)
    for g in range(ngrp):
        z = accs[g] * inv
        res_v[pl.ds(g * 16, 16)] = 1.0 / (1.0 + jnp.exp(-z))
    pltpu.sync_copy(res_v, out_hbm.at[pl.ds(wid * ROWS_PER_W, ROWS_PER_W)])


def kernel(x, table, W, b):
    table2 = table.reshape(MV_ROWS, 128)
    w = W[:, 0]
    wtile = (
        jnp.zeros((128, 2), jnp.float32)
        .at[0:64, 0].set(w)
        .at[64:128, 1].set(w)
    )
    b2 = b.reshape(1, 1)
    tw = _tw_matvec(table2, wtile, b2).reshape(VOCAB)

    x3 = x.astype(jnp.int32).reshape(NW, ROWS_PER_W, SEQ)
    xr = _idx_transpose(x3).reshape(NW, SEQ * ROWS_PER_W)
    out = _sc_pool(xr, tw)
    return out.reshape(BATCH, 1)


# speedup vs baseline: 1.2862x; 1.2862x over previous
import functools

import jax
import jax.numpy as jnp
from jax import lax
from jax.experimental import pallas as pl
from jax.experimental.pallas import tpu as pltpu
from jax.experimental.pallas import tpu_sc as plsc

VOCAB = 1_000_000
VOCAB_PAD = 1 << 20
EMBED = 64
BATCH = 4096
SEQ = 200

NC, NS = 2, 16
NW = NC * NS
ROWS_PER_W = BATCH // NW
IDX_PER_W = ROWS_PER_W * SEQ

T2_ROWS = VOCAB // 2
MV_GRID = 128
MV_IN_BLK = 4096
T2_LAST_BLK = (T2_ROWS - 1) // MV_IN_BLK


def _matvec_body(t_ref, w_ref, d_ref, b_ref, o_ref):
    tmp = jnp.dot(t_ref[...], w_ref[...], preferred_element_type=jnp.float32)
    t3 = tmp.reshape(64, 64, 128)
    sel = jnp.where(d_ref[...][None] != 0.0, t3, 0.0)
    o_ref[...] = jnp.sum(sel, axis=1) + b_ref[0, 0]


def _tw_matvec(table2, wtile, dmask, b2):
    return pl.pallas_call(
        _matvec_body,
        grid=(MV_GRID,),
        in_specs=[
            pl.BlockSpec(
                (MV_IN_BLK, 128),
                lambda i: (jnp.minimum(i, T2_LAST_BLK), 0),
            ),
            pl.BlockSpec((128, 128), lambda i: (0, 0)),
            pl.BlockSpec((64, 128), lambda i: (0, 0)),
            pl.BlockSpec((1, 1), lambda i: (0, 0)),
        ],
        out_specs=pl.BlockSpec((64, 128), lambda i: (i, 0)),
        out_shape=jax.ShapeDtypeStruct((VOCAB_PAD // 128, 128), jnp.float32),
    )(table2, wtile, dmask, b2)


def _transpose_body(x_ref, o_ref):
    o_ref[0] = jnp.swapaxes(x_ref[0], 0, 1)


def _idx_transpose(x3):
    return pl.pallas_call(
        _transpose_body,
        grid=(NW,),
        in_specs=[pl.BlockSpec((1, ROWS_PER_W, SEQ), lambda i: (i, 0, 0))],
        out_specs=pl.BlockSpec((1, SEQ, ROWS_PER_W), lambda i: (i, 0, 0)),
        out_shape=jax.ShapeDtypeStruct((NW, SEQ, ROWS_PER_W), jnp.int32),
    )(x3)


_SC_MESH = plsc.VectorSubcoreMesh(core_axis_name="c", subcore_axis_name="s")


@functools.partial(
    pl.kernel,
    out_type=jax.ShapeDtypeStruct((BATCH,), jnp.float32),
    mesh=_SC_MESH,
    scratch_types=[
        pltpu.VMEM((IDX_PER_W,), jnp.int32),
        pltpu.VMEM((IDX_PER_W,), jnp.float32),
        pltpu.VMEM((ROWS_PER_W,), jnp.float32),
        pltpu.SemaphoreType.DMA,
    ],
)
def _sc_pool(xr_hbm, tw_hbm, out_hbm, idx_v, vals_v, res_v, sem):
    wid = lax.axis_index("s") * NC + lax.axis_index("c")
    pltpu.sync_copy(xr_hbm.at[pl.ds(wid * IDX_PER_W, IDX_PER_W)], idx_v)
    pltpu.async_copy(tw_hbm.at[idx_v], vals_v, sem).wait()

    ngrp = ROWS_PER_W // 16

    def seq_body(j, accs):
        base = j * ROWS_PER_W
        return tuple(
            accs[g] + vals_v[pl.ds(base + g * 16, 16)] for g in range(ngrp)
        )

    accs = lax.fori_loop(
        0, SEQ, seq_body, tuple(jnp.zeros((16,), jnp.float32) for _ in range(ngrp))
    )
    inv = jnp.float32(1.0 / SEQ)
    for g in range(ngrp):
        z = accs[g] * inv
        res_v[pl.ds(g * 16, 16)] = 1.0 / (1.0 + jnp.exp(-z))
    pltpu.sync_copy(res_v, out_hbm.at[pl.ds(wid * ROWS_PER_W, ROWS_PER_W)])


def kernel(x, table, W, b):
    table2 = table.reshape(T2_ROWS, 128)
    w = W[:, 0]
    k_half = lax.iota(jnp.int32, 128)[:, None] // 64
    l_par = lax.iota(jnp.int32, 128)[None, :] % 2
    wtile = jnp.where(k_half == l_par, jnp.tile(w, 2)[:, None], 0.0)
    dmask = jnp.where(
        lax.iota(jnp.int32, 64)[:, None]
        == lax.iota(jnp.int32, 128)[None, :] // 2,
        1.0,
        0.0,
    ).astype(jnp.float32)
    b2 = b.reshape(1, 1)
    tw = _tw_matvec(table2, wtile, dmask, b2).reshape(VOCAB_PAD)

    x3 = x.astype(jnp.int32).reshape(NW, ROWS_PER_W, SEQ)
    xr = _idx_transpose(x3).reshape(NW * IDX_PER_W)
    out = _sc_pool(xr, tw)
    return out.reshape(BATCH, 1)


# speedup vs baseline: 6.8517x; 5.3271x over previous
import functools

import jax
import jax.numpy as jnp
from jax import lax
from jax.experimental import pallas as pl
from jax.experimental.pallas import tpu as pltpu
from jax.experimental.pallas import tpu_sc as plsc

VOCAB = 1_000_000
VOCAB_PAD = 1 << 20
EMBED = 64
BATCH = 4096
SEQ = 200

NC, NS = 2, 16
NW = NC * NS
ROWS_PER_W = BATCH // NW
IDX_PER_W = ROWS_PER_W * SEQ

MV_SUB = 8192
MV_OUT_ROWS = 8
MV_QUARTERS = 4
MV_IN_COLS = MV_OUT_ROWS * MV_SUB // MV_QUARTERS
MV_GRID = VOCAB_PAD // (MV_OUT_ROWS * MV_SUB)


def _matvec_body(t_ref, w_ref, b_ref, o_ref):
    q = pl.program_id(1)
    rows_per_q = MV_OUT_ROWS // MV_QUARTERS
    for r in range(rows_per_q):
        row = q * rows_per_q + r
        o_ref[row, :] = (
            jnp.dot(
                w_ref[...],
                t_ref[:, pl.ds(r * MV_SUB, MV_SUB)],
                preferred_element_type=jnp.float32,
            )[0]
            + b_ref[0, 0]
        )


def _tw_matvec(table_t, w_row, b2):
    return pl.pallas_call(
        _matvec_body,
        grid=(MV_GRID, MV_QUARTERS),
        in_specs=[
            pl.BlockSpec(
                (EMBED, MV_IN_COLS),
                lambda i, q: (
                    0,
                    jnp.minimum(
                        i * MV_QUARTERS + q, (VOCAB - 1) // MV_IN_COLS
                    ),
                ),
            ),
            pl.BlockSpec((1, EMBED), lambda i, q: (0, 0)),
            pl.BlockSpec((1, 1), lambda i, q: (0, 0)),
        ],
        out_specs=pl.BlockSpec((MV_OUT_ROWS, MV_SUB), lambda i, q: (i, 0)),
        out_shape=jax.ShapeDtypeStruct(
            (VOCAB_PAD // MV_SUB, MV_SUB), jnp.float32
        ),
    )(table_t, w_row, b2)


_SC_MESH = plsc.VectorSubcoreMesh(core_axis_name="c", subcore_axis_name="s")


@functools.partial(
    pl.kernel,
    out_type=jax.ShapeDtypeStruct((BATCH,), jnp.float32),
    mesh=_SC_MESH,
    scratch_types=[
        pltpu.VMEM((IDX_PER_W,), jnp.int32),
        pltpu.VMEM((IDX_PER_W,), jnp.float32),
        pltpu.VMEM((ROWS_PER_W,), jnp.float32),
        pltpu.SemaphoreType.DMA,
    ],
)
def _sc_pool(xr_hbm, tw_hbm, out_hbm, idx_v, vals_v, res_v, sem):
    wid = lax.axis_index("s") * NC + lax.axis_index("c")
    pltpu.sync_copy(xr_hbm.at[pl.ds(wid * IDX_PER_W, IDX_PER_W)], idx_v)
    pltpu.async_copy(tw_hbm.at[idx_v], vals_v, sem).wait()

    ngrp = ROWS_PER_W // 16

    def seq_body(j, accs):
        base = j * ROWS_PER_W
        return tuple(
            accs[g] + vals_v[pl.ds(base + g * 16, 16)] for g in range(ngrp)
        )

    accs = lax.fori_loop(
        0, SEQ, seq_body, tuple(jnp.zeros((16,), jnp.float32) for _ in range(ngrp))
    )
    inv = jnp.float32(1.0 / SEQ)
    for g in range(ngrp):
        z = accs[g] * inv
        res_v[pl.ds(g * 16, 16)] = 1.0 / (1.0 + jnp.exp(-z))
    pltpu.sync_copy(res_v, out_hbm.at[pl.ds(wid * ROWS_PER_W, ROWS_PER_W)])


def kernel(x, table, W, b):
    table_t = table.T
    w_row = W.T
    b2 = b.reshape(1, 1)
    tw = _tw_matvec(table_t, w_row, b2).reshape(VOCAB_PAD)

    xr = (
        jnp.swapaxes(x.astype(jnp.int32).T.reshape(SEQ, NW, ROWS_PER_W), 0, 1)
        .reshape(NW * IDX_PER_W)
    )
    out = _sc_pool(xr, tw)
    return out.reshape(BATCH, 1)
